# baseline (speedup 1.0000x reference)
import functools

import jax
import jax.numpy as jnp
from jax import lax
from jax.experimental import pallas as pl
from jax.experimental.pallas import tpu as pltpu
from jax.experimental.pallas import tpu_sc as plsc

_R = 128
_V = 100000
_NC = 2
_NS = 16
_NW = _NC * _NS
_RPW = _R // _NW
_CHUNK = 10000
_NCH = _V // _CHUNK
_VPC = _CHUNK // 16


@functools.lru_cache(maxsize=1)
def _gumbel_flat():
    key = jax.random.key(42)
    u = jax.random.uniform(key, (_R, _V), dtype=jnp.float32,
                           minval=1e-20, maxval=1.0)
    g = -jnp.log(-jnp.log(u))
    return jax.block_until_ready(g.reshape(_R * _V))


def _shuffle(x, idx):
    dn = lax.GatherDimensionNumbers(
        offset_dims=(), collapsed_slice_dims=(0,), start_index_map=(0,))
    return lax.gather(x, idx[:, None], dn, slice_sizes=(1,),
                      mode=lax.GatherScatterMode.PROMISE_IN_BOUNDS)


def _sc_argmax(x_flat, g_flat):
    mesh = plsc.VectorSubcoreMesh(core_axis_name="c", subcore_axis_name="s")

    @functools.partial(
        pl.kernel,
        out_type=jax.ShapeDtypeStruct((_NW, 16), jnp.int32),
        mesh=mesh,
        scratch_types=[
            pltpu.VMEM((_CHUNK,), jnp.float32),
            pltpu.VMEM((_CHUNK,), jnp.float32),
            pltpu.VMEM((16,), jnp.int32),
        ],
    )
    def k(x_hbm, g_hbm, out_hbm, xbuf, gbuf, outv):
        wid = lax.axis_index("s") * _NC + lax.axis_index("c")
        base = wid * (_RPW * _V)
        lane = lax.broadcasted_iota(jnp.int32, (16,), 0)
        res = jnp.zeros((16,), jnp.int32)
        for r in range(_RPW):
            m = jnp.full((16,), -jnp.inf, jnp.float32)
            bi = jnp.zeros((16,), jnp.int32)
            for c in range(_NCH):
                off = base + r * _V + c * _CHUNK
                pltpu.sync_copy(x_hbm.at[pl.ds(off, _CHUNK)], xbuf)
                pltpu.sync_copy(g_hbm.at[pl.ds(off, _CHUNK)], gbuf)

                def body(j, carry):
                    mm, bb, ci = carry
                    v = xbuf[pl.ds(j * 16, 16)] + gbuf[pl.ds(j * 16, 16)]
                    upd = v > mm
                    mm = jnp.where(upd, v, mm)
                    bb = jnp.where(upd, ci, bb)
                    return mm, bb, ci + 16

                ci0 = lane + c * _CHUNK
                m, bi, _ = lax.fori_loop(0, _VPC, body, (m, bi, ci0))
            for s in (8, 4, 2, 1):
                perm = lane ^ s
                pm = _shuffle(m, perm)
                pbi = _shuffle(bi, perm)
                take = (pm > m) | ((pm == m) & (pbi < bi))
                m = jnp.where(take, pm, m)
                bi = jnp.where(take, pbi, bi)
            res = jnp.where(lane == r, bi, res)
        outv[...] = res
        pltpu.sync_copy(outv, out_hbm.at[wid])

    return k(x_flat, g_flat)


def kernel(logits):
    g = _gumbel_flat()
    out2d = _sc_argmax(logits.reshape(_R * _V), g)
    return out2d[:, :_RPW].reshape(_R).astype(jnp.int64)


# speedup vs baseline: 1.1958x; 1.1958x over previous
import functools

import jax
import jax.numpy as jnp
from jax import lax
from jax.experimental import pallas as pl
from jax.experimental.pallas import tpu as pltpu
from jax.experimental.pallas import tpu_sc as plsc

_R = 128
_V = 100000
_NC = 2
_NS = 16
_NW = _NC * _NS
_RPW = _R // _NW
_L = 16
_CHUNK = 20000
_CPR = _V // _CHUNK
_NT = _RPW * _CPR
_U = 10
_VPC = _CHUNK // _L
_IT = _VPC // _U


@functools.lru_cache(maxsize=1)
def _gumbel_flat():
    key = jax.random.key(42)
    u = jax.random.uniform(key, (_R, _V), dtype=jnp.float32,
                           minval=1e-20, maxval=1.0)
    g = -jnp.log(-jnp.log(u))
    return jax.block_until_ready(g.reshape(_R * _V))


def _shuffle(x, idx):
    dn = lax.GatherDimensionNumbers(
        offset_dims=(), collapsed_slice_dims=(0,), start_index_map=(0,))
    return lax.gather(x, idx[:, None], dn, slice_sizes=(1,),
                      mode=lax.GatherScatterMode.PROMISE_IN_BOUNDS)


def _merge(m_a, bi_a, m_b, bi_b):
    take = (m_b > m_a) | ((m_b == m_a) & (bi_b < bi_a))
    return jnp.where(take, m_b, m_a), jnp.where(take, bi_b, bi_a)


def _sc_argmax(x_flat, g_flat):
    mesh = plsc.VectorSubcoreMesh(core_axis_name="c", subcore_axis_name="s")

    @functools.partial(
        pl.kernel,
        out_type=jax.ShapeDtypeStruct((_NW, _L), jnp.int32),
        mesh=mesh,
        scratch_types=[
            pltpu.VMEM((_CHUNK,), jnp.float32),
            pltpu.VMEM((_CHUNK,), jnp.float32),
            pltpu.VMEM((_CHUNK,), jnp.float32),
            pltpu.VMEM((_CHUNK,), jnp.float32),
            pltpu.VMEM((_L,), jnp.int32),
            pltpu.SemaphoreType.DMA,
            pltpu.SemaphoreType.DMA,
            pltpu.SemaphoreType.DMA,
            pltpu.SemaphoreType.DMA,
        ],
    )
    def k(x_hbm, g_hbm, out_hbm, xbuf0, xbuf1, gbuf0, gbuf1, outv,
          sx0, sx1, sg0, sg1):
        wid = lax.axis_index("s") * _NC + lax.axis_index("c")
        base = wid * (_RPW * _V)
        lane = lax.broadcasted_iota(jnp.int32, (_L,), 0)
        xbufs = (xbuf0, xbuf1)
        gbufs = (gbuf0, gbuf1)
        sx = (sx0, sx1)
        sg = (sg0, sg1)

        def start(t):
            b = t % 2
            off = base + t * _CHUNK
            cx = pltpu.async_copy(x_hbm.at[pl.ds(off, _CHUNK)], xbufs[b],
                                  sx[b])
            cg = pltpu.async_copy(g_hbm.at[pl.ds(off, _CHUNK)], gbufs[b],
                                  sg[b])
            return cx, cg

        pending = start(0)
        res = jnp.zeros((_L,), jnp.int32)
        for r in range(_RPW):
            ms = [jnp.full((_L,), -jnp.inf, jnp.float32)] * _U
            bis = [jnp.zeros((_L,), jnp.int32)] * _U
            for c in range(_CPR):
                t = r * _CPR + c
                b = t % 2
                nxt = start(t + 1) if t + 1 < _NT else None
                pending[0].wait()
                pending[1].wait()
                if nxt is not None:
                    pending = nxt
                xb = xbufs[b]
                gb = gbufs[b]

                def body(j, carry, _c=c, _xb=xb, _gb=gb):
                    mm = list(carry[:_U])
                    bb = list(carry[_U:2 * _U])
                    cb = carry[2 * _U]
                    o = j * (_U * _L)
                    for u in range(_U):
                        v = (_xb[pl.ds(o + u * _L, _L)]
                             + _gb[pl.ds(o + u * _L, _L)])
                        ci = cb + (u * _L) if u else cb
                        upd = v > mm[u]
                        mm[u] = jnp.where(upd, v, mm[u])
                        bb[u] = jnp.where(upd, ci, bb[u])
                    return (*mm, *bb, cb + _U * _L)

                cb0 = lane + c * _CHUNK
                carry = lax.fori_loop(0, _IT, body, (*ms, *bis, cb0))
                ms = list(carry[:_U])
                bis = list(carry[_U:2 * _U])
            m, bi = ms[0], bis[0]
            for u in range(1, _U):
                m, bi = _merge(m, bi, ms[u], bis[u])
            for s in (8, 4, 2, 1):
                perm = lane ^ s
                m, bi = _merge(m, bi, _shuffle(m, perm), _shuffle(bi, perm))
            res = jnp.where(lane == r, bi, res)
        outv[...] = res
        pltpu.sync_copy(outv, out_hbm.at[wid])

    return k(x_flat, g_flat)


def kernel(logits):
    g = _gumbel_flat()
    out2d = _sc_argmax(logits.reshape(_R * _V), g)
    return out2d[:, :_RPW].reshape(_R).astype(jnp.int64)


# speedup vs baseline: 5.3937x; 4.5105x over previous
import functools

import jax
import jax.numpy as jnp
from jax import lax
from jax.experimental import pallas as pl
from jax.experimental.pallas import tpu as pltpu
from jax.experimental.pallas import tpu_sc as plsc

_R = 128
_V = 100000
_NC = 2
_NS = 16
_NW = _NC * _NS
_L = 16
_TPH = 390
_CT = 13
_NCHK = 30
_CW = _CT * 128
_VTAIL = 2 * _TPH * 128


@functools.lru_cache(maxsize=1)
def _gumbel2d():
    with jax.ensure_compile_time_eval():
        key = jax.random.key(42)
        u = jax.random.uniform(key, (_R, _V), dtype=jnp.float32,
                               minval=1e-20, maxval=1.0)
        g = -jnp.log(-jnp.log(u))
    return jax.block_until_ready(g)


def _merge(m_a, bi_a, m_b, bi_b):
    take = (m_b > m_a) | ((m_b == m_a) & (bi_b < bi_a))
    return jnp.where(take, m_b, m_a), jnp.where(take, bi_b, bi_a)


def _shuffle(x, idx):
    dn = lax.GatherDimensionNumbers(
        offset_dims=(), collapsed_slice_dims=(0,), start_index_map=(0,))
    return lax.gather(x, idx[:, None], dn, slice_sizes=(1,),
                      mode=lax.GatherScatterMode.PROMISE_IN_BOUNDS)


def _sc_argmax(x):
    mesh = plsc.VectorSubcoreMesh(core_axis_name="c", subcore_axis_name="s")

    @functools.partial(
        pl.kernel,
        out_type=(jax.ShapeDtypeStruct((_NW, _L), jnp.int32),
                  jax.ShapeDtypeStruct((_NW, _L), jnp.float32)),
        mesh=mesh,
        scratch_types=[
            pltpu.VMEM((8, _CW), jnp.float32),
            pltpu.VMEM((8, _CW), jnp.float32),
            pltpu.VMEM((_L,), jnp.int32),
            pltpu.VMEM((_L,), jnp.float32),
            pltpu.SemaphoreType.DMA,
            pltpu.SemaphoreType.DMA,
        ],
    )
    def k(x_hbm, oi_hbm, om_hbm, xb0, xb1, oiv, omv, sx0, sx1):
        c = lax.axis_index("c")
        s = lax.axis_index("s")
        wid = s * _NC + c
        grp = c * 8 + s // 2
        h = s % 2
        r0 = grp * 8
        cbase = h * (_TPH * 128)
        lane = lax.broadcasted_iota(jnp.int32, (_L,), 0)
        xbufs = (xb0, xb1)
        sx = (sx0, sx1)

        def start(t):
            b = t % 2
            c0 = cbase + t * _CW
            return pltpu.async_copy(
                x_hbm.at[pl.ds(r0, 8), pl.ds(c0, _CW)], xbufs[b], sx[b])

        pending = start(0)
        ms = [jnp.full((_L,), -jnp.inf, jnp.float32) for _ in range(8)]
        bis = [jnp.zeros((_L,), jnp.int32) for _ in range(8)]
        for t in range(_NCHK):
            b = t % 2
            nxt = start(t + 1) if t + 1 < _NCHK else None
            pending.wait()
            if nxt is not None:
                pending = nxt
            xb = xbufs[b]

            def body(j, carry, _xb=xb):
                mm = list(carry[:8])
                bb = list(carry[8:16])
                ci0 = carry[16]
                o = j * 64
                cis = [ci0 + (i * _L) if i else ci0 for i in range(4)]
                for sl in range(8):
                    for i in range(4):
                        v = _xb[sl, pl.ds(o + i * _L, _L)]
                        upd = v > mm[sl]
                        mm[sl] = jnp.where(upd, v, mm[sl])
                        bb[sl] = jnp.where(upd, cis[i], bb[sl])
                return (*mm, *bb, ci0 + 64)

            ci0 = lane + (cbase + t * _CW)
            carry = lax.fori_loop(0, 2 * _CT, body, (*ms, *bis, ci0))
            ms = list(carry[:8])
            bis = list(carry[8:16])
        resm = jnp.full((_L,), -jnp.inf, jnp.float32)
        resi = jnp.zeros((_L,), jnp.int32)
        for sl in range(8):
            m, bi = ms[sl], bis[sl]
            for st in (8, 4, 2, 1):
                perm = lane ^ st
                m, bi = _merge(m, bi, _shuffle(m, perm), _shuffle(bi, perm))
            resm = jnp.where(lane == sl, m, resm)
            resi = jnp.where(lane == sl, bi, resi)
        oiv[...] = resi
        omv[...] = resm
        pltpu.sync_copy(oiv, oi_hbm.at[wid])
        pltpu.sync_copy(omv, om_hbm.at[wid])

    return k(x)


def kernel(logits):
    sv = logits + _gumbel2d()
    oi, om = _sc_argmax(sv)
    wid = jnp.arange(_NW, dtype=jnp.int32)
    c = wid % _NC
    s = wid // _NC
    key = (c * 8 + s // 2) * 2 + (s % 2)
    order = jnp.argsort(key)
    mi = om[order].reshape(16, 2, _L)[:, :, :8]
    ii = oi[order].reshape(16, 2, _L)[:, :, :8]
    m0, i0 = mi[:, 0].reshape(_R), ii[:, 0].reshape(_R)
    m1, i1 = mi[:, 1].reshape(_R), ii[:, 1].reshape(_R)
    mm, im = _merge(m0, i0, m1, i1)
    tail = sv[:, _VTAIL:]
    tm = jnp.max(tail, axis=-1)
    ti = jnp.argmax(tail, axis=-1).astype(jnp.int32) + _VTAIL
    mm, im = _merge(mm, im, tm, ti)
    return im.astype(jnp.int64)
